# baseline (speedup 1.0000x reference)
import functools

import jax
import jax.numpy as jnp
from jax import lax
from jax.experimental import pallas as pl
from jax.experimental.pallas import tpu as pltpu
from jax.experimental.pallas import tpu_sc as plsc

_IDX_CHUNK = 128


def _sc_gather3(W, V, iu2d, ib2d, iw2d, bpw, n_chunks, K):
    B = iu2d.shape[0] * _IDX_CHUNK
    mesh = plsc.VectorSubcoreMesh(core_axis_name="c", subcore_axis_name="s")
    info = plsc.get_sparse_core_info()
    NC = info.num_cores

    out_t = (jax.ShapeDtypeStruct((B, K), jnp.float32),) * 3
    scratch_t = [
        pltpu.VMEM((n_chunks, _IDX_CHUNK), jnp.int32),
        pltpu.VMEM((n_chunks, _IDX_CHUNK), jnp.int32),
        pltpu.VMEM((n_chunks, _IDX_CHUNK), jnp.int32),
        pltpu.VMEM((bpw, K), jnp.float32),
        pltpu.VMEM((bpw, K), jnp.float32),
        pltpu.VMEM((bpw, K), jnp.float32),
        pltpu.SemaphoreType.DMA,
    ]

    @functools.partial(
        pl.kernel, mesh=mesh, out_type=out_t, scratch_types=scratch_t,
        compiler_params=pltpu.CompilerParams(use_tc_tiling_on_sc=False))
    def k(w_hbm, v_hbm, iu_hbm, ib_hbm, iw_hbm, ou_hbm, ob_hbm, ow_hbm,
          iu_v, ib_v, iw_v, ru_v, rb_v, rw_v, sem):
        wid = lax.axis_index("s") * NC + lax.axis_index("c")
        ibase = wid * n_chunks
        rbase = wid * bpw
        pltpu.sync_copy(iu_hbm.at[pl.ds(ibase, n_chunks)], iu_v)
        pltpu.sync_copy(ib_hbm.at[pl.ds(ibase, n_chunks)], ib_v)
        pltpu.sync_copy(iw_hbm.at[pl.ds(ibase, n_chunks)], iw_v)
        for j in range(n_chunks):
            dst = pl.ds(j * _IDX_CHUNK, _IDX_CHUNK)
            pltpu.async_copy(w_hbm.at[iu_v.at[j]], ru_v.at[dst], sem)
            pltpu.async_copy(v_hbm.at[ib_v.at[j]], rb_v.at[dst], sem)
            pltpu.async_copy(v_hbm.at[iw_v.at[j]], rw_v.at[dst], sem)
        for j in range(n_chunks):
            dst = pl.ds(j * _IDX_CHUNK, _IDX_CHUNK)
            pltpu.make_async_copy(w_hbm.at[iu_v.at[j]], ru_v.at[dst], sem).wait()
            pltpu.make_async_copy(v_hbm.at[ib_v.at[j]], rb_v.at[dst], sem).wait()
            pltpu.make_async_copy(v_hbm.at[iw_v.at[j]], rw_v.at[dst], sem).wait()
        pltpu.sync_copy(ru_v, ou_hbm.at[pl.ds(rbase, bpw)])
        pltpu.sync_copy(rb_v, ob_hbm.at[pl.ds(rbase, bpw)])
        pltpu.sync_copy(rw_v, ow_hbm.at[pl.ds(rbase, bpw)])

    return k(W, V, iu2d, ib2d, iw2d)


def _tc_outer(wu, vb, vw, BB, K):
    B = wu.shape[0]
    KK = K * K

    def body(wu_ref, vb_ref, vw_ref, fb_ref, fw_ref):
        col = lax.broadcasted_iota(jnp.int32, (K, KK), 1)
        row = lax.broadcasted_iota(jnp.int32, (K, KK), 0)
        rep_m = (col // K == row).astype(jnp.float32)
        til_m = (col % K == row).astype(jnp.float32)
        rep = jnp.dot(wu_ref[...], rep_m, preferred_element_type=jnp.float32)
        fb_ref[...] = rep * jnp.dot(vb_ref[...], til_m,
                                    preferred_element_type=jnp.float32)
        fw_ref[...] = rep * jnp.dot(vw_ref[...], til_m,
                                    preferred_element_type=jnp.float32)

    in_spec = pl.BlockSpec((BB, K), lambda i: (i, 0))
    out_spec = pl.BlockSpec((BB, KK), lambda i: (i, 0))
    return pl.pallas_call(
        body,
        grid=(B // BB,),
        in_specs=[in_spec, in_spec, in_spec],
        out_specs=[out_spec, out_spec],
        out_shape=[jax.ShapeDtypeStruct((B, KK), jnp.float32)] * 2,
    )(wu, vb, vw)


def kernel(input_user, best_movie, worst_movie, W, V):
    B = input_user.shape[0]
    K = W.shape[1]
    iu = input_user.reshape(B // _IDX_CHUNK, _IDX_CHUNK).astype(jnp.int32)
    ib = best_movie.reshape(B // _IDX_CHUNK, _IDX_CHUNK).astype(jnp.int32)
    iw = worst_movie.reshape(B // _IDX_CHUNK, _IDX_CHUNK).astype(jnp.int32)

    info = plsc.get_sparse_core_info()
    nw = info.num_cores * info.num_subcores
    bpw = B // nw
    n_chunks = bpw // _IDX_CHUNK

    wu, vb, vw = _sc_gather3(W, V, iu, ib, iw, bpw, n_chunks, K)
    fb, fw = _tc_outer(wu, vb, vw, BB=512, K=K)
    return fb.reshape(B, K, K), fw.reshape(B, K, K)
